# baseline (speedup 1.0000x reference)
import functools

import jax
import jax.numpy as jnp
from jax import lax
from jax.experimental import pallas as pl
from jax.experimental.pallas import tpu as pltpu
from jax.experimental.pallas import tpu_sc as plsc

EMB = 32
ROW = EMB + 1
VOCAB_WORDS = 1000000 * ROW
VROW = 16
NVROW = VOCAB_WORDS // VROW
WIN = 3 * VROW
NC, NS = 2, 16
NW = NC * NS
CHUNK = 128
B = 4096 * 26
PER_W = B // NW
NCHUNK = PER_W // CHUNK
L = 16
NG = CHUNK // L


def _make_kernel():
  mesh = plsc.VectorSubcoreMesh(core_axis_name="c", subcore_axis_name="s")

  @functools.partial(
      pl.kernel,
      out_type=(
          jax.ShapeDtypeStruct((B * EMB,), jnp.float32),
          jax.ShapeDtypeStruct((B,), jnp.float32),
      ),
      mesh=mesh,
      compiler_params=pltpu.CompilerParams(
          needs_layout_passes=False, use_tc_tiling_on_sc=False
      ),
      scratch_types=[
          pltpu.VMEM((NCHUNK, CHUNK), jnp.int32),
          pltpu.VMEM((3, CHUNK), jnp.int32),
          pltpu.VMEM((CHUNK,), jnp.int32),
          pltpu.VMEM((3 * CHUNK, VROW), jnp.float32),
          pltpu.VMEM((PER_W * EMB,), jnp.float32),
          pltpu.VMEM((PER_W,), jnp.float32),
          pltpu.SemaphoreType.DMA,
      ],
  )
  def k(idx_hbm, tab_hbm, w_hbm, b_hbm,
        idx_v, q_v, rho_v, win_v, w_v, b_v, sem):
    wid = lax.axis_index("s") * NC + lax.axis_index("c")
    base = wid * PER_W

    pltpu.sync_copy(idx_hbm.at[wid], idx_v)

    lanes = lax.iota(jnp.int32, L)

    def chunk(g, carry):
      def prep(t, c):
        idx16 = idx_v[g, pl.ds(t * L, L)]
        start = idx16 * ROW
        r0 = lax.shift_right_logical(start, 4)
        q_v[0, pl.ds(t * L, L)] = r0
        q_v[1, pl.ds(t * L, L)] = r0 + 1
        q_v[2, pl.ds(t * L, L)] = r0 + 2
        rho_v[pl.ds(t * L, L)] = lax.bitwise_and(start, 15)
        return c

      lax.fori_loop(0, NG, prep, 0)

      for j in range(3):
        pltpu.make_async_copy(
            tab_hbm.at[q_v.at[j]],
            win_v.at[pl.ds(j * CHUNK, CHUNK)],
            sem,
        ).start()
      for j in range(3):
        pltpu.make_async_copy(
            tab_hbm.at[q_v.at[j]],
            win_v.at[pl.ds(j * CHUNK, CHUNK)],
            sem,
        ).wait()

      def win_idx(r16, w16):
        row = lax.shift_left(lax.shift_right_logical(w16, 4), 7) + r16
        return [row, lax.bitwise_and(w16, 15)]

      def extract(t, c):
        r16 = lanes + t * L
        rho = rho_v[pl.ds(t * L, L)]
        dst0 = (g * CHUNK + t * L + lanes) * EMB
        for col in range(EMB):
          vals = plsc.load_gather(win_v, win_idx(r16, rho + col))
          plsc.store_scatter(w_v, [dst0 + col], vals)
        bias = plsc.load_gather(win_v, win_idx(r16, rho + EMB))
        b_v[pl.ds(g * CHUNK + t * L, L)] = bias
        return c

      lax.fori_loop(0, NG, extract, 0)
      return carry

    lax.fori_loop(0, NCHUNK, chunk, 0)

    pltpu.sync_copy(w_v, w_hbm.at[pl.ds(base * EMB, PER_W * EMB)])
    pltpu.sync_copy(b_v, b_hbm.at[pl.ds(base, PER_W)])

  return k


_gather = _make_kernel()


@jax.jit
def kernel(input, table):
  idx = input.astype(jnp.int32).reshape(NW, NCHUNK, CHUNK)
  tab16 = table.reshape(NVROW, VROW)
  w_flat, b_flat = _gather(idx, tab16)
  w = w_flat.reshape(*input.shape, EMB)
  b = b_flat.reshape(input.shape)
  return (w, b)


# speedup vs baseline: 1.6686x; 1.6686x over previous
import functools

import jax
import jax.numpy as jnp
from jax import lax
from jax.experimental import pallas as pl
from jax.experimental.pallas import tpu as pltpu
from jax.experimental.pallas import tpu_sc as plsc

EMB = 32
VOCAB = 1000000
NC, NS = 2, 16
NW = NC * NS
CHUNK = 128
B = 4096 * 26
PER_W = B // NW
NCHUNK = PER_W // CHUNK
L = 16
BPITCH = 8


def _make_gather_kernel():
  mesh = plsc.VectorSubcoreMesh(core_axis_name="c", subcore_axis_name="s")

  @functools.partial(
      pl.kernel,
      out_type=(
          jax.ShapeDtypeStruct((B, EMB), jnp.float32),
          jax.ShapeDtypeStruct((B,), jnp.float32),
      ),
      mesh=mesh,
      compiler_params=pltpu.CompilerParams(
          needs_layout_passes=False, use_tc_tiling_on_sc=False
      ),
      scratch_types=[
          pltpu.VMEM((NCHUNK, CHUNK), jnp.int32),
          pltpu.VMEM((2, CHUNK), jnp.int32),
          pltpu.VMEM((PER_W, EMB), jnp.float32),
          pltpu.VMEM((2, CHUNK, BPITCH), jnp.float32),
          pltpu.VMEM((PER_W,), jnp.float32),
          pltpu.SemaphoreType.DMA,
          pltpu.SemaphoreType.DMA,
      ],
  )
  def kb(idx_hbm, wt_hbm, bt_hbm, w_hbm, b_hbm,
         idx_v, q_v, w_v, b8_v, b_v, sem_w, sem_b):
    wid = lax.axis_index("s") * NC + lax.axis_index("c")
    base = wid * PER_W

    pltpu.sync_copy(idx_hbm.at[wid], idx_v)

    lanes = lax.iota(jnp.int32, L)

    def prep(g, buf):
      def grp(t8, c2):
        idx16 = idx_v[g, pl.ds(t8 * L, L)]
        q_v[buf, pl.ds(t8 * L, L)] = lax.shift_right_logical(idx16, 3)
        return c2

      lax.fori_loop(0, CHUNK // L, grp, 0)

    def fire(g, buf):
      pltpu.make_async_copy(
          wt_hbm.at[idx_v.at[g]],
          w_v.at[pl.ds(g * CHUNK, CHUNK)],
          sem_w,
      ).start()
      pltpu.make_async_copy(
          bt_hbm.at[q_v.at[buf]],
          b8_v.at[buf],
          sem_b,
      ).start()

    def drain(g, buf):
      pltpu.make_async_copy(
          wt_hbm.at[idx_v.at[g]],
          w_v.at[pl.ds(g * CHUNK, CHUNK)],
          sem_w,
      ).wait()
      pltpu.make_async_copy(
          bt_hbm.at[q_v.at[buf]],
          b8_v.at[buf],
          sem_b,
      ).wait()

    def extract(g, buf):
      bufv = jnp.full((L,), buf, jnp.int32)

      def grp(t8, c2):
        idx16 = idx_v[g, pl.ds(t8 * L, L)]
        vals = plsc.load_gather(
            b8_v, [bufv, lanes + t8 * L, lax.bitwise_and(idx16, 7)]
        )
        b_v[pl.ds(g * CHUNK + t8 * L, L)] = vals
        return c2

      lax.fori_loop(0, CHUNK // L, grp, 0)

    prep(0, 0)
    fire(0, 0)

    def pair(g2, carry):
      g0 = g2 * 2
      prep(g0 + 1, 1)
      fire(g0 + 1, 1)
      drain(g0, 0)
      extract(g0, 0)

      @pl.when(g0 + 2 < NCHUNK)
      def _():
        prep(g0 + 2, 0)
        fire(g0 + 2, 0)

      drain(g0 + 1, 1)
      extract(g0 + 1, 1)
      return carry

    lax.fori_loop(0, NCHUNK // 2, pair, 0)

    pltpu.sync_copy(w_v, w_hbm.at[pl.ds(base, PER_W)])
    pltpu.sync_copy(b_v, b_hbm.at[pl.ds(base, PER_W)])

  return kb


_gather = _make_gather_kernel()


@jax.jit
def kernel(input, table):
  idx = input.astype(jnp.int32).reshape(NW, NCHUNK, CHUNK)
  wt = table[:, :EMB].reshape(-1).reshape(VOCAB, EMB)
  bt = table[:, EMB].reshape(VOCAB // BPITCH, BPITCH)
  w_flat, b_flat = _gather(idx, wt, bt)
  w = w_flat.reshape(*input.shape, EMB)
  b = b_flat.reshape(input.shape)
  return (w, b)
